# baseline (speedup 1.0000x reference)
import functools

import jax
import jax.numpy as jnp
from jax import lax
from jax.experimental import pallas as pl
from jax.experimental.pallas import tpu as pltpu
from jax.experimental.pallas import tpu_sc as plsc

_B = 4096
_F = 26
_E = 16
_D = _F * _E
_BF = _B * _F

_NC = 2
_NS = 16
_NW = _NC * _NS
_CHUNK = 128
_NCHUNK = _BF // (_NW * _CHUNK)
_SLAB = _NCHUNK * _CHUNK * _E // 128


def _sc_gather_body(idx_hbm, tlr_hbm, tdp_hbm, wide_hbm, deep_hbm,
                    idx_v, rows_lr, rows_dp, pl0, pd0, pl1, pd1,
                    sem_lr, sem_dp, wsem):
    wid = lax.axis_index("s") * _NC + lax.axis_index("c")
    pltpu.sync_copy(idx_hbm.at[wid], idx_v)

    def issue(j, carry):
        pltpu.async_copy(tlr_hbm.at[idx_v.at[j]], rows_lr.at[j], sem_lr)
        pltpu.async_copy(tdp_hbm.at[idx_v.at[j]], rows_dp.at[j], sem_dp)
        return carry

    lax.fori_loop(0, _NCHUNK, issue, 0)
    pltpu.make_async_copy(wide_hbm.at[wid], rows_lr, sem_lr).wait()
    pltpu.make_async_copy(deep_hbm.at[wid], rows_dp, sem_dp).wait()

    paks = ((pl0, pd0), (pl1, pd1))

    def repack(jj, carry):
        for b in range(2):
            j = 2 * jj + b
            pkl, pkd = paks[b]

            @pl.when(jj > 0)
            def _():
                pltpu.make_async_copy(
                    wide_hbm.at[wid, pl.ds(0, 16)], pkl, wsem).wait()
                pltpu.make_async_copy(
                    wide_hbm.at[wid, pl.ds(0, 16)], pkd, wsem).wait()

            for r in range(16):
                for c in range(8):
                    pkl[r, pl.ds(c * 16, 16)] = rows_lr[j, 8 * r + c, :]
                    pkd[r, pl.ds(c * 16, 16)] = rows_dp[j, 8 * r + c, :]
            pltpu.async_copy(pkl, wide_hbm.at[wid, pl.ds(j * 16, 16)], wsem)
            pltpu.async_copy(pkd, deep_hbm.at[wid, pl.ds(j * 16, 16)], wsem)
        return carry

    lax.fori_loop(0, _NCHUNK // 2, repack, 0)
    for b in range(2):
        pkl, pkd = paks[b]
        pltpu.make_async_copy(wide_hbm.at[wid, pl.ds(0, 16)], pkl, wsem).wait()
        pltpu.make_async_copy(wide_hbm.at[wid, pl.ds(0, 16)], pkd, wsem).wait()


@functools.cache
def _make_sc_gather():
    return pl.kernel(
        _sc_gather_body,
        out_type=[
            jax.ShapeDtypeStruct((_NW, _SLAB, 128), jnp.float32),
            jax.ShapeDtypeStruct((_NW, _SLAB, 128), jnp.float32),
        ],
        mesh=plsc.VectorSubcoreMesh(core_axis_name="c", subcore_axis_name="s"),
        compiler_params=pltpu.CompilerParams(
            use_tc_tiling_on_sc=False, needs_layout_passes=False),
        scratch_types=[
            pltpu.VMEM((_NCHUNK, _CHUNK), jnp.int32),
            pltpu.VMEM((_NCHUNK, _CHUNK, _E), jnp.float32),
            pltpu.VMEM((_NCHUNK, _CHUNK, _E), jnp.float32),
            pltpu.VMEM((16, 128), jnp.float32),
            pltpu.VMEM((16, 128), jnp.float32),
            pltpu.VMEM((16, 128), jnp.float32),
            pltpu.VMEM((16, 128), jnp.float32),
            pltpu.SemaphoreType.DMA,
            pltpu.SemaphoreType.DMA,
            pltpu.SemaphoreType.DMA,
        ],
    )


_BLK = 512
_NBLK = _B // _BLK


def _stats_body(deep_ref, acc_ref):
    i = pl.program_id(0)
    blk = deep_ref[...]
    s = jnp.sum(blk, axis=0, keepdims=True)
    q = jnp.sum(blk * blk, axis=0, keepdims=True)
    sq = jnp.concatenate([s, q], axis=0)

    @pl.when(i == 0)
    def _():
        acc_ref[...] = sq

    @pl.when(i != 0)
    def _():
        acc_ref[...] += sq


def _mlp_body(stats_ref, gamma_ref, beta_ref, deep_ref, wide_ref,
              w1_ref, b1_ref, w2_ref, b2_ref, w3_ref, b3_ref, out_ref):
    inv_b = 1.0 / _B
    mean = stats_ref[0:1, :] * inv_b
    var = stats_ref[1:2, :] * inv_b - mean * mean
    scale = gamma_ref[...] * lax.rsqrt(var + 1e-5)
    shift = beta_ref[...] - mean * scale
    h = (deep_ref[...] * scale + shift).astype(jnp.bfloat16)
    h1 = jnp.maximum(
        jnp.dot(h, w1_ref[...], preferred_element_type=jnp.float32)
        + b1_ref[...], 0.0).astype(jnp.bfloat16)
    h2 = jnp.maximum(
        jnp.dot(h1, w2_ref[...], preferred_element_type=jnp.float32)
        + b2_ref[...], 0.0)
    d = jnp.sum(h2 * w3_ref[...], axis=1, keepdims=True) + b3_ref[...]
    out_ref[...] = jax.nn.sigmoid(wide_ref[...] + d)


def _tc_stats(deep):
    return pl.pallas_call(
        _stats_body,
        grid=(_NBLK,),
        in_specs=[pl.BlockSpec((_BLK, _D), lambda i: (i, 0))],
        out_specs=pl.BlockSpec((2, _D), lambda i: (0, 0)),
        out_shape=jax.ShapeDtypeStruct((2, _D), jnp.float32),
    )(deep)


def _tc_mlp(stats, gamma, beta, deep, wide, w1, b1, w2, b2, w3, b3):
    fixed = lambda i: (0, 0)
    return pl.pallas_call(
        _mlp_body,
        grid=(_NBLK,),
        in_specs=[
            pl.BlockSpec((2, _D), fixed),
            pl.BlockSpec((1, _D), fixed),
            pl.BlockSpec((1, _D), fixed),
            pl.BlockSpec((_BLK, _D), lambda i: (i, 0)),
            pl.BlockSpec((_BLK, _D), lambda i: (i, 0)),
            pl.BlockSpec((_D, 1024), fixed),
            pl.BlockSpec((1, 1024), fixed),
            pl.BlockSpec((1024, 512), fixed),
            pl.BlockSpec((1, 512), fixed),
            pl.BlockSpec((1, 512), fixed),
            pl.BlockSpec((1, 1), fixed),
        ],
        out_specs=pl.BlockSpec((_BLK, _D), lambda i: (i, 0)),
        out_shape=jax.ShapeDtypeStruct((_B, _D), jnp.float32),
    )(stats, gamma, beta, deep, wide, w1, b1, w2, b2, w3, b3)


def kernel(x, table_lr, table_deep, gamma, beta, W1, b1, W2, b2, W3, b3):
    idx = x.astype(jnp.int32).reshape(_NW, _NCHUNK, _CHUNK)
    wide3, deep3 = _make_sc_gather()(idx, table_lr, table_deep)
    wide = wide3.reshape(_B, _D)
    deep = deep3.reshape(_B, _D)
    stats = _tc_stats(deep)
    return _tc_mlp(stats, gamma.reshape(1, _D), beta.reshape(1, _D),
                   deep, wide, W1.astype(jnp.bfloat16), b1.reshape(1, 1024),
                   W2.astype(jnp.bfloat16), b2.reshape(1, 512),
                   W3.reshape(1, 512), b3.reshape(1, 1))


# speedup vs baseline: 1.0111x; 1.0111x over previous
import functools

import jax
import jax.numpy as jnp
from jax import lax
from jax.experimental import pallas as pl
from jax.experimental.pallas import tpu as pltpu
from jax.experimental.pallas import tpu_sc as plsc

_B = 4096
_F = 26
_E = 16
_D = _F * _E
_DP = 512
_BF = _B * _F

_NC = 2
_NS = 16
_NW = _NC * _NS
_ROWS_W = _B // _NW
_CB = 4
_CIDX = _CB * _F
_NCHUNK = _ROWS_W // _CB
_IDX_W = _ROWS_W * _F
_LINES_W = _ROWS_W * _DP // 128


def _sc_gather_body(idx_hbm, tlr_hbm, tdp_hbm, wide_hbm, deep_hbm,
                    idx_v, rows_lr, rows_dp, pl0, pd0, pl1, pd1,
                    sem_lr, sem_dp, wsem):
    wid = lax.axis_index("s") * _NC + lax.axis_index("c")
    pltpu.sync_copy(idx_hbm.at[wid], idx_v)

    def issue(cc, carry):
        pltpu.async_copy(tlr_hbm.at[idx_v.at[cc]],
                         rows_lr.at[pl.ds(cc * _CIDX, _CIDX)], sem_lr)
        pltpu.async_copy(tdp_hbm.at[idx_v.at[cc]],
                         rows_dp.at[pl.ds(cc * _CIDX, _CIDX)], sem_dp)
        return carry

    lax.fori_loop(0, _NCHUNK, issue, 0)
    pltpu.make_async_copy(tlr_hbm.at[pl.ds(0, _IDX_W)], rows_lr, sem_lr).wait()
    pltpu.make_async_copy(tlr_hbm.at[pl.ds(0, _IDX_W)], rows_dp, sem_dp).wait()

    paks = ((pl0, pd0), (pl1, pd1))

    def repack(jj, carry):
        for b in range(2):
            cc = 2 * jj + b
            pkl, pkd = paks[b]

            @pl.when(jj > 0)
            def _():
                pltpu.make_async_copy(
                    wide_hbm.at[wid, pl.ds(0, 16)], pkl, wsem).wait()
                pltpu.make_async_copy(
                    wide_hbm.at[wid, pl.ds(0, 16)], pkd, wsem).wait()

            base = cc * _CIDX
            for p in range(_CIDX):
                line = (p // _F) * 4 + (p % _F) // 8
                off = ((p % _F) % 8) * 16
                pkl[line, pl.ds(off, 16)] = rows_lr[base + p, :]
                pkd[line, pl.ds(off, 16)] = rows_dp[base + p, :]
            pltpu.async_copy(pkl, wide_hbm.at[wid, pl.ds(cc * 16, 16)], wsem)
            pltpu.async_copy(pkd, deep_hbm.at[wid, pl.ds(cc * 16, 16)], wsem)
        return carry

    lax.fori_loop(0, _NCHUNK // 2, repack, 0)
    for b in range(2):
        pkl, pkd = paks[b]
        pltpu.make_async_copy(wide_hbm.at[wid, pl.ds(0, 16)], pkl, wsem).wait()
        pltpu.make_async_copy(wide_hbm.at[wid, pl.ds(0, 16)], pkd, wsem).wait()


@functools.cache
def _make_sc_gather():
    return pl.kernel(
        _sc_gather_body,
        out_type=[
            jax.ShapeDtypeStruct((_NW, _LINES_W, 128), jnp.float32),
            jax.ShapeDtypeStruct((_NW, _LINES_W, 128), jnp.float32),
        ],
        mesh=plsc.VectorSubcoreMesh(core_axis_name="c", subcore_axis_name="s"),
        compiler_params=pltpu.CompilerParams(
            use_tc_tiling_on_sc=False, needs_layout_passes=False),
        scratch_types=[
            pltpu.VMEM((_NCHUNK, _CIDX), jnp.int32),
            pltpu.VMEM((_IDX_W, _E), jnp.float32),
            pltpu.VMEM((_IDX_W, _E), jnp.float32),
            pltpu.VMEM((16, 128), jnp.float32),
            pltpu.VMEM((16, 128), jnp.float32),
            pltpu.VMEM((16, 128), jnp.float32),
            pltpu.VMEM((16, 128), jnp.float32),
            pltpu.SemaphoreType.DMA,
            pltpu.SemaphoreType.DMA,
            pltpu.SemaphoreType.DMA,
        ],
    )


_BLK = 512
_NBLK = _B // _BLK


def _stats_body(deep_ref, acc_ref):
    i = pl.program_id(0)
    blk = deep_ref[:, :_D]
    s = jnp.sum(blk, axis=0, keepdims=True)
    q = jnp.sum(blk * blk, axis=0, keepdims=True)
    sq = jnp.concatenate([s, q], axis=0)

    @pl.when(i == 0)
    def _():
        acc_ref[...] = sq

    @pl.when(i != 0)
    def _():
        acc_ref[...] += sq


def _mlp_body(stats_ref, gamma_ref, beta_ref, deep_ref, wide_ref,
              w1_ref, b1_ref, w2_ref, b2_ref, w3_ref, b3_ref, out_ref):
    inv_b = 1.0 / _B
    mean = stats_ref[0:1, :] * inv_b
    var = stats_ref[1:2, :] * inv_b - mean * mean
    scale = gamma_ref[...] * lax.rsqrt(var + 1e-5)
    shift = beta_ref[...] - mean * scale
    h = (deep_ref[:, :_D] * scale + shift).astype(jnp.bfloat16)
    h1 = jnp.maximum(
        jnp.dot(h, w1_ref[...], preferred_element_type=jnp.float32)
        + b1_ref[...], 0.0).astype(jnp.bfloat16)
    h2 = jnp.maximum(
        jnp.dot(h1, w2_ref[...], preferred_element_type=jnp.float32)
        + b2_ref[...], 0.0)
    d = jnp.sum(h2 * w3_ref[...], axis=1, keepdims=True) + b3_ref[...]
    out_ref[...] = jax.nn.sigmoid(wide_ref[:, :_D] + d)


def _tc_stats(deep):
    return pl.pallas_call(
        _stats_body,
        grid=(_NBLK,),
        in_specs=[pl.BlockSpec((_BLK, _DP), lambda i: (i, 0))],
        out_specs=pl.BlockSpec((2, _D), lambda i: (0, 0)),
        out_shape=jax.ShapeDtypeStruct((2, _D), jnp.float32),
    )(deep)


def _tc_mlp(stats, gamma, beta, deep, wide, w1, b1, w2, b2, w3, b3):
    fixed = lambda i: (0, 0)
    return pl.pallas_call(
        _mlp_body,
        grid=(_NBLK,),
        in_specs=[
            pl.BlockSpec((2, _D), fixed),
            pl.BlockSpec((1, _D), fixed),
            pl.BlockSpec((1, _D), fixed),
            pl.BlockSpec((_BLK, _DP), lambda i: (i, 0)),
            pl.BlockSpec((_BLK, _DP), lambda i: (i, 0)),
            pl.BlockSpec((_D, 1024), fixed),
            pl.BlockSpec((1, 1024), fixed),
            pl.BlockSpec((1024, 512), fixed),
            pl.BlockSpec((1, 512), fixed),
            pl.BlockSpec((1, 512), fixed),
            pl.BlockSpec((1, 1), fixed),
        ],
        out_specs=pl.BlockSpec((_BLK, _D), lambda i: (i, 0)),
        out_shape=jax.ShapeDtypeStruct((_B, _D), jnp.float32),
    )(stats, gamma, beta, deep, wide, w1, b1, w2, b2, w3, b3)


def kernel(x, table_lr, table_deep, gamma, beta, W1, b1, W2, b2, W3, b3):
    idx = x.astype(jnp.int32).reshape(_NW, _NCHUNK, _CIDX)
    wide3, deep3 = _make_sc_gather()(idx, table_lr, table_deep)
    wide = wide3.reshape(_B, _DP)
    deep = deep3.reshape(_B, _DP)
    stats = _tc_stats(deep)
    return _tc_mlp(stats, gamma.reshape(1, _D), beta.reshape(1, _D),
                   deep, wide, W1.astype(jnp.bfloat16), b1.reshape(1, 1024),
                   W2.astype(jnp.bfloat16), b2.reshape(1, 512),
                   W3.reshape(1, 512), b3.reshape(1, 1))


# speedup vs baseline: 1.1691x; 1.1563x over previous
import functools

import jax
import jax.numpy as jnp
from jax import lax
from jax.experimental import pallas as pl
from jax.experimental.pallas import tpu as pltpu
from jax.experimental.pallas import tpu_sc as plsc

_B = 4096
_F = 26
_E = 16
_D = _F * _E
_DP = 512
_BF = _B * _F

_NC = 2
_NS = 16
_NW = _NC * _NS
_ROWS_W = _B // _NW
_CB = 4
_CIDX = _CB * _F
_NCHUNK = _ROWS_W // _CB
_IDX_W = _ROWS_W * _F
_LINES_W = _ROWS_W * _DP // 128


def _sc_gather_body(idx_hbm, tbl_hbm, out_hbm,
                    idx_v, rows, pk0, pk1, sem, wsem):
    wid = lax.axis_index("s") * _NC + lax.axis_index("c")
    pltpu.sync_copy(idx_hbm.at[wid], idx_v)

    def issue(cc, carry):
        pltpu.async_copy(tbl_hbm.at[idx_v.at[cc]],
                         rows.at[pl.ds(cc * _CIDX, _CIDX)], sem)
        return carry

    lax.fori_loop(0, _NCHUNK, issue, 0)
    pltpu.make_async_copy(tbl_hbm.at[pl.ds(0, _IDX_W)], rows, sem).wait()

    paks = (pk0, pk1)

    def repack(jj, carry):
        for b in range(2):
            cc = 2 * jj + b
            pk = paks[b]

            @pl.when(jj > 0)
            def _():
                pltpu.make_async_copy(
                    out_hbm.at[wid, pl.ds(0, 16)], pk, wsem).wait()

            base = cc * _CIDX
            for p in range(_CIDX):
                line = (p // _F) * 4 + (p % _F) // 8
                off = ((p % _F) % 8) * 16
                pk[line, pl.ds(off, 16)] = rows[base + p, :]
            pltpu.async_copy(pk, out_hbm.at[wid, pl.ds(cc * 16, 16)], wsem)
        return carry

    lax.fori_loop(0, _NCHUNK // 2, repack, 0)
    for b in range(2):
        pltpu.make_async_copy(
            out_hbm.at[wid, pl.ds(0, 16)], paks[b], wsem).wait()


@functools.cache
def _make_sc_gather():
    return pl.kernel(
        _sc_gather_body,
        out_type=jax.ShapeDtypeStruct((_NW, _LINES_W, 128), jnp.float32),
        mesh=plsc.VectorSubcoreMesh(core_axis_name="c", subcore_axis_name="s"),
        compiler_params=pltpu.CompilerParams(
            use_tc_tiling_on_sc=False, needs_layout_passes=False),
        scratch_types=[
            pltpu.VMEM((_NCHUNK, _CIDX), jnp.int32),
            pltpu.VMEM((_IDX_W, _E), jnp.float32),
            pltpu.VMEM((16, 128), jnp.float32),
            pltpu.VMEM((16, 128), jnp.float32),
            pltpu.SemaphoreType.DMA,
            pltpu.SemaphoreType.DMA,
        ],
    )


_BLK = 512
_NBLK = _B // _BLK


def _stats_body(deep_ref, acc_ref):
    i = pl.program_id(0)
    blk = deep_ref[:, :_D]
    s = jnp.sum(blk, axis=0, keepdims=True)
    q = jnp.sum(blk * blk, axis=0, keepdims=True)
    sq = jnp.concatenate([s, q], axis=0)

    @pl.when(i == 0)
    def _():
        acc_ref[...] = sq

    @pl.when(i != 0)
    def _():
        acc_ref[...] += sq


def _mlp_body(stats_ref, gamma_ref, beta_ref, deep_ref, wide_ref,
              w1_ref, b1_ref, w2_ref, b2_ref, w3_ref, b3_ref, out_ref):
    inv_b = 1.0 / _B
    mean = stats_ref[0:1, :] * inv_b
    var = stats_ref[1:2, :] * inv_b - mean * mean
    scale = gamma_ref[...] * lax.rsqrt(var + 1e-5)
    shift = beta_ref[...] - mean * scale
    h = (deep_ref[:, :_D] * scale + shift).astype(jnp.bfloat16)
    h1 = jnp.maximum(
        jnp.dot(h, w1_ref[...], preferred_element_type=jnp.float32)
        + b1_ref[...], 0.0).astype(jnp.bfloat16)
    h2 = jnp.maximum(
        jnp.dot(h1, w2_ref[...], preferred_element_type=jnp.float32)
        + b2_ref[...], 0.0)
    d = jnp.sum(h2 * w3_ref[...], axis=1, keepdims=True) + b3_ref[...]
    out_ref[...] = jax.nn.sigmoid(wide_ref[:, :_D] + d)


def _tc_stats(deep):
    return pl.pallas_call(
        _stats_body,
        grid=(_NBLK,),
        in_specs=[pl.BlockSpec((_BLK, _DP), lambda i: (i, 0))],
        out_specs=pl.BlockSpec((2, _D), lambda i: (0, 0)),
        out_shape=jax.ShapeDtypeStruct((2, _D), jnp.float32),
    )(deep)


def _tc_mlp(stats, gamma, beta, deep, wide, w1, b1, w2, b2, w3, b3):
    fixed = lambda i: (0, 0)
    return pl.pallas_call(
        _mlp_body,
        grid=(_NBLK,),
        in_specs=[
            pl.BlockSpec((2, _D), fixed),
            pl.BlockSpec((1, _D), fixed),
            pl.BlockSpec((1, _D), fixed),
            pl.BlockSpec((_BLK, _DP), lambda i: (i, 0)),
            pl.BlockSpec((_BLK, _DP), lambda i: (i, 0)),
            pl.BlockSpec((_D, 1024), fixed),
            pl.BlockSpec((1, 1024), fixed),
            pl.BlockSpec((1024, 512), fixed),
            pl.BlockSpec((1, 512), fixed),
            pl.BlockSpec((1, 512), fixed),
            pl.BlockSpec((1, 1), fixed),
        ],
        out_specs=pl.BlockSpec((_BLK, _D), lambda i: (i, 0)),
        out_shape=jax.ShapeDtypeStruct((_B, _D), jnp.float32),
    )(stats, gamma, beta, deep, wide, w1, b1, w2, b2, w3, b3)


def kernel(x, table_lr, table_deep, gamma, beta, W1, b1, W2, b2, W3, b3):
    idx = x.astype(jnp.int32).reshape(_NW, _NCHUNK, _CIDX)
    gather = _make_sc_gather()
    deep3 = gather(idx, table_deep)
    wide3 = gather(idx, table_lr)
    wide = wide3.reshape(_B, _DP)
    deep = deep3.reshape(_B, _DP)
    stats = _tc_stats(deep)
    return _tc_mlp(stats, gamma.reshape(1, _D), beta.reshape(1, _D),
                   deep, wide, W1.astype(jnp.bfloat16), b1.reshape(1, 1024),
                   W2.astype(jnp.bfloat16), b2.reshape(1, 512),
                   W3.reshape(1, 512), b3.reshape(1, 1))


# speedup vs baseline: 1.2722x; 1.0881x over previous
import functools

import jax
import jax.numpy as jnp
from jax import lax
from jax.experimental import pallas as pl
from jax.experimental.pallas import tpu as pltpu
from jax.experimental.pallas import tpu_sc as plsc

_B = 4096
_F = 26
_E = 16
_D = _F * _E
_DP = 512
_BF = _B * _F

_NC = 2
_NS = 16
_NW = _NC * _NS
_ROWS_W = _B // _NW
_CB = 4
_CIDX = _CB * _F
_NCHUNK = _ROWS_W // _CB
_IDX_W = _ROWS_W * _F
_LINES_W = _ROWS_W * _DP // 128


def _sc_gather_body(idx_hbm, tbl_hbm, out_hbm,
                    idx_v, rows, pk0, pk1, sem, wsem):
    wid = lax.axis_index("s") * _NC + lax.axis_index("c")
    pltpu.sync_copy(idx_hbm.at[wid], idx_v)

    def issue(cc, carry):
        pltpu.async_copy(tbl_hbm.at[idx_v.at[cc]],
                         rows.at[pl.ds(cc * _CIDX, _CIDX)], sem)
        return carry

    lax.fori_loop(0, _NCHUNK, issue, 0)
    pltpu.make_async_copy(tbl_hbm.at[pl.ds(0, _IDX_W)], rows, sem).wait()

    paks = (pk0, pk1)

    def repack(jj, carry):
        for b in range(2):
            cc = 2 * jj + b
            pk = paks[b]

            @pl.when(jj > 0)
            def _():
                pltpu.make_async_copy(
                    out_hbm.at[wid, pl.ds(0, 16)], pk, wsem).wait()

            base = cc * _CIDX
            for p in range(_CIDX):
                line = (p // _F) * 4 + (p % _F) // 8
                off = ((p % _F) % 8) * 16
                pk[line, pl.ds(off, 16)] = rows[base + p, :]
            pltpu.async_copy(pk, out_hbm.at[wid, pl.ds(cc * 16, 16)], wsem)
        return carry

    lax.fori_loop(0, _NCHUNK // 2, repack, 0)
    for b in range(2):
        pltpu.make_async_copy(
            out_hbm.at[wid, pl.ds(0, 16)], paks[b], wsem).wait()


@functools.cache
def _make_sc_gather():
    return pl.kernel(
        _sc_gather_body,
        out_type=jax.ShapeDtypeStruct((_NW, _LINES_W, 128), jnp.float32),
        mesh=plsc.VectorSubcoreMesh(core_axis_name="c", subcore_axis_name="s"),
        compiler_params=pltpu.CompilerParams(
            use_tc_tiling_on_sc=False, needs_layout_passes=False),
        scratch_types=[
            pltpu.VMEM((_NCHUNK, _CIDX), jnp.int32),
            pltpu.VMEM((_IDX_W, _E), jnp.float32),
            pltpu.VMEM((16, 128), jnp.float32),
            pltpu.VMEM((16, 128), jnp.float32),
            pltpu.SemaphoreType.DMA,
            pltpu.SemaphoreType.DMA,
        ],
    )


_BLK = 512
_NBLK = _B // _BLK
_WPB = _BLK // _ROWS_W


def _stats_body(deep_ref, acc_ref):
    i = pl.program_id(0)
    blk = deep_ref[...].reshape(_BLK, _DP)[:, :_D]
    s = jnp.sum(blk, axis=0, keepdims=True)
    q = jnp.sum(blk * blk, axis=0, keepdims=True)
    sq = jnp.concatenate([s, q], axis=0)

    @pl.when(i == 0)
    def _():
        acc_ref[...] = sq

    @pl.when(i != 0)
    def _():
        acc_ref[...] += sq


def _mlp_body(stats_ref, gamma_ref, beta_ref, deep_ref, wide_ref,
              w1_ref, b1_ref, w2_ref, b2_ref, w3_ref, b3_ref, out_ref):
    inv_b = 1.0 / _B
    mean = stats_ref[0:1, :] * inv_b
    var = stats_ref[1:2, :] * inv_b - mean * mean
    scale = gamma_ref[...] * lax.rsqrt(var + 1e-5)
    shift = beta_ref[...] - mean * scale
    deep = deep_ref[...].reshape(_BLK, _DP)
    h = (deep[:, :_D] * scale + shift).astype(jnp.bfloat16)
    h1 = jnp.maximum(
        jnp.dot(h, w1_ref[...], preferred_element_type=jnp.float32)
        + b1_ref[...], 0.0).astype(jnp.bfloat16)
    h2 = jnp.maximum(
        jnp.dot(h1, w2_ref[...], preferred_element_type=jnp.float32)
        + b2_ref[...], 0.0)
    d = jnp.sum(h2 * w3_ref[...], axis=1, keepdims=True) + b3_ref[...]
    wide = wide_ref[...].reshape(_BLK, _DP)
    out_ref[...] = jax.nn.sigmoid(wide[:, :_D] + d)


def _tc_stats(deep):
    return pl.pallas_call(
        _stats_body,
        grid=(_NBLK,),
        in_specs=[pl.BlockSpec((_WPB, _LINES_W, 128), lambda i: (i, 0, 0))],
        out_specs=pl.BlockSpec((2, _D), lambda i: (0, 0)),
        out_shape=jax.ShapeDtypeStruct((2, _D), jnp.float32),
    )(deep)


def _tc_mlp(stats, gamma, beta, deep, wide, w1, b1, w2, b2, w3, b3):
    fixed = lambda i: (0, 0)
    return pl.pallas_call(
        _mlp_body,
        grid=(_NBLK,),
        in_specs=[
            pl.BlockSpec((2, _D), fixed),
            pl.BlockSpec((1, _D), fixed),
            pl.BlockSpec((1, _D), fixed),
            pl.BlockSpec((_WPB, _LINES_W, 128), lambda i: (i, 0, 0)),
            pl.BlockSpec((_WPB, _LINES_W, 128), lambda i: (i, 0, 0)),
            pl.BlockSpec((_D, 1024), fixed),
            pl.BlockSpec((1, 1024), fixed),
            pl.BlockSpec((1024, 512), fixed),
            pl.BlockSpec((1, 512), fixed),
            pl.BlockSpec((1, 512), fixed),
            pl.BlockSpec((1, 1), fixed),
        ],
        out_specs=pl.BlockSpec((_BLK, _D), lambda i: (i, 0)),
        out_shape=jax.ShapeDtypeStruct((_B, _D), jnp.float32),
    )(stats, gamma, beta, deep, wide, w1, b1, w2, b2, w3, b3)


def kernel(x, table_lr, table_deep, gamma, beta, W1, b1, W2, b2, W3, b3):
    idx = x.astype(jnp.int32).reshape(_NW, _NCHUNK, _CIDX)
    gather = _make_sc_gather()
    deep3 = gather(idx, table_deep)
    wide3 = gather(idx, table_lr)
    stats = _tc_stats(deep3)
    return _tc_mlp(stats, gamma.reshape(1, _D), beta.reshape(1, _D),
                   deep3, wide3, W1.astype(jnp.bfloat16), b1.reshape(1, 1024),
                   W2.astype(jnp.bfloat16), b2.reshape(1, 512),
                   W3.reshape(1, 512), b3.reshape(1, 1))
